# baseline (speedup 1.0000x reference)
import dataclasses
import functools

import jax
import jax.numpy as jnp
from jax import lax
from jax.experimental import pallas as pl
from jax.experimental.pallas import tpu as pltpu
from jax.experimental.pallas import tpu_sc as plsc

BATCH = 4096
MAX_LEN = 200
EMBED_DIM = 32
NUM_WORKERS = 32
EPW = BATCH // NUM_WORKERS
IDX_CHUNK = 100
LANES = 16
NBUF = 8
ROUNDS = EPW // NBUF
UNROLL = 4
TRANS_BLOCK = 8192
PAD_LANES = 128


def _tc_format_bf16(emb_t):
    vocab = emb_t.shape[1]
    grid = pl.cdiv(vocab, TRANS_BLOCK)

    def body(x_ref, o_ref):
        o_ref[:, :EMBED_DIM] = x_ref[...].T.astype(jnp.bfloat16)

    return pl.pallas_call(
        body,
        grid=(grid,),
        in_specs=[pl.BlockSpec((EMBED_DIM, TRANS_BLOCK), lambda i: (0, i))],
        out_specs=pl.BlockSpec((TRANS_BLOCK, PAD_LANES), lambda i: (i, 0)),
        out_shape=jax.ShapeDtypeStruct((vocab, PAD_LANES), jnp.bfloat16),
    )(emb_t)


def _sc_classify(xr4, emb_rows, w_even, w_odd):
    mesh = plsc.VectorSubcoreMesh(core_axis_name="c", subcore_axis_name="s")
    cp = pltpu.CompilerParams()
    for field, val in (("needs_layout_passes", False),
                       ("use_tc_tiling_on_sc", False)):
        if field in pltpu.CompilerParams.__dataclass_fields__:
            cp = dataclasses.replace(cp, **{field: val})

    @functools.partial(
        pl.kernel,
        out_type=jax.ShapeDtypeStruct((BATCH,), jnp.float32),
        mesh=mesh,
        compiler_params=cp,
        scratch_types=(
            [pltpu.VMEM((2 * EPW, IDX_CHUNK), jnp.int32)]
            + [pltpu.VMEM((MAX_LEN, EMBED_DIM), jnp.bfloat16)
               for _ in range(NBUF)]
            + [pltpu.VMEM((MAX_LEN, LANES), jnp.float32)
               for _ in range(2)]
            + [pltpu.VMEM((EPW,), jnp.float32)]
            + [pltpu.SemaphoreType.DMA for _ in range(NBUF)]
        ),
    )
    def k(x_hbm, emb_hbm, we_hbm, wo_hbm, out_hbm, idx_v, *scr):
        rows = scr[:NBUF]
        we_v = scr[NBUF]
        wo_v = scr[NBUF + 1]
        acc_v = scr[NBUF + 2]
        sems = scr[NBUF + 3:]
        wid = lax.axis_index("c") * 16 + lax.axis_index("s")
        pltpu.sync_copy(x_hbm.at[pl.ds(wid * (2 * EPW), 2 * EPW)], idx_v)
        pltpu.sync_copy(we_hbm, we_v)
        pltpu.sync_copy(wo_hbm, wo_v)
        lanes = lax.iota(jnp.int32, LANES)

        def issue(e, b):
            pltpu.async_copy(emb_hbm.at[idx_v.at[2 * e]],
                             rows[b].at[pl.ds(0, IDX_CHUNK)], sems[b])
            pltpu.async_copy(emb_hbm.at[idx_v.at[2 * e + 1]],
                             rows[b].at[pl.ds(IDX_CHUNK, IDX_CHUNK)], sems[b])

        def wait(b):
            pltpu.make_async_copy(emb_hbm.at[pl.ds(0, IDX_CHUNK)],
                                  rows[b].at[pl.ds(0, IDX_CHUNK)],
                                  sems[b]).wait()
            pltpu.make_async_copy(emb_hbm.at[pl.ds(0, IDX_CHUNK)],
                                  rows[b].at[pl.ds(IDX_CHUNK, IDX_CHUNK)],
                                  sems[b]).wait()

        for b in range(NBUF):
            issue(b, b)

        def round_body(r, res):
            for b in range(NBUF):
                elem = r * NBUF + b
                wait(b)

                @pl.when(r < ROUNDS - 1)
                def _():
                    issue(elem + NBUF, b)

                def body(l4, accs):
                    a0, a1 = accs
                    for kk in range(UNROLL):
                        l = l4 * UNROLL + kk
                        packed = plsc.bitcast(rows[b][l, :], jnp.int32)
                        lo = plsc.bitcast(packed << 16, jnp.float32)
                        hi = plsc.bitcast(packed & jnp.int32(-65536), jnp.float32)
                        a0 = a0 + lo * we_v[l, :]
                        a1 = a1 + hi * wo_v[l, :]
                    return (a0, a1)

                a0, a1 = lax.fori_loop(
                    0, MAX_LEN // UNROLL, body,
                    (jnp.zeros(LANES, jnp.float32),
                     jnp.zeros(LANES, jnp.float32)))
                s = jnp.sum(a0 + a1)
                res = jnp.where(lanes == (r % 2) * NBUF + b, s, res)

            @pl.when(r % 2 == 1)
            def _():
                off = pl.multiple_of((r // 2) * (2 * NBUF), 2 * NBUF)
                acc_v[pl.ds(off, LANES)] = res

            return jnp.where(r % 2 == 1, jnp.zeros_like(res), res)

        lax.fori_loop(0, ROUNDS, round_body, jnp.zeros(LANES, jnp.float32))
        pltpu.sync_copy(acc_v, out_hbm.at[pl.ds(wid * EPW, EPW)])

    return k(xr4, emb_rows, w_even, w_odd)


@jax.jit
def kernel(x, emb_table, W, b):
    xr4 = (x.astype(jnp.int32) * 4).reshape(2 * BATCH, IDX_CHUNK)
    w2 = W.reshape(MAX_LEN, EMBED_DIM)
    w_even = w2[:, 0::2]
    w_odd = w2[:, 1::2]
    emb_bf = _tc_format_bf16(emb_table.T)
    emb_view = emb_bf.reshape(4 * emb_table.shape[0], EMBED_DIM)
    out = _sc_classify(xr4, emb_view, w_even, w_odd)
    return out.reshape(BATCH, 1) + b


# speedup vs baseline: 2.4808x; 2.4808x over previous
import dataclasses
import functools

import jax
import jax.numpy as jnp
from jax import lax
from jax.experimental import pallas as pl
from jax.experimental.pallas import tpu as pltpu
from jax.experimental.pallas import tpu_sc as plsc

BATCH = 4096
MAX_LEN = 200
EMBED_DIM = 32
NUM_WORKERS = 32
EPW = BATCH // NUM_WORKERS
IDX_CHUNK = 100
LANES = 16
NBUF = 8
ROUNDS = EPW // NBUF
UNROLL = 4
TRANS_BLOCK = 8192
PAD_LANES = 128


def _tc_format_bf16(emb_t):
    vocab = emb_t.shape[1]
    grid = pl.cdiv(vocab, TRANS_BLOCK)
    packed_rows = TRANS_BLOCK // 8
    half = EMBED_DIM // 2

    def body(x_ref, o_ref, t_ref):
        y = lax.bitcast_convert_type(x_ref[...].T, jnp.uint32)
        lo = (y[:, :half] + jnp.uint32(0x8000)) >> 16
        hi = (y[:, half:] + jnp.uint32(0x8000)) & jnp.uint32(0xFFFF0000)
        t_ref[...] = lo | hi
        o_ref[...] = jnp.concatenate(
            [t_ref[pl.Slice(k, packed_rows, 8), :] for k in range(8)], axis=1)

    return pl.pallas_call(
        body,
        grid=(grid,),
        in_specs=[pl.BlockSpec((EMBED_DIM, TRANS_BLOCK), lambda i: (0, i))],
        out_specs=pl.BlockSpec((packed_rows, 8 * half), lambda i: (i, 0)),
        out_shape=jax.ShapeDtypeStruct((vocab // 8, 8 * half), jnp.uint32),
        scratch_shapes=[pltpu.VMEM((TRANS_BLOCK, half), jnp.uint32)],
    )(emb_t)


def _sc_classify(xr4, emb_rows, w_even, w_odd):
    mesh = plsc.VectorSubcoreMesh(core_axis_name="c", subcore_axis_name="s")
    cp = pltpu.CompilerParams()
    for field, val in (("needs_layout_passes", False),
                       ("use_tc_tiling_on_sc", False)):
        if field in pltpu.CompilerParams.__dataclass_fields__:
            cp = dataclasses.replace(cp, **{field: val})

    @functools.partial(
        pl.kernel,
        out_type=jax.ShapeDtypeStruct((BATCH,), jnp.float32),
        mesh=mesh,
        compiler_params=cp,
        scratch_types=(
            [pltpu.VMEM((2 * EPW, IDX_CHUNK), jnp.int32)]
            + [pltpu.VMEM((MAX_LEN, LANES), jnp.uint32)
               for _ in range(NBUF)]
            + [pltpu.VMEM((MAX_LEN, LANES), jnp.float32)
               for _ in range(2)]
            + [pltpu.VMEM((EPW,), jnp.float32)]
            + [pltpu.SemaphoreType.DMA for _ in range(NBUF)]
        ),
    )
    def k(x_hbm, emb_hbm, we_hbm, wo_hbm, out_hbm, idx_v, *scr):
        rows = scr[:NBUF]
        we_v = scr[NBUF]
        wo_v = scr[NBUF + 1]
        acc_v = scr[NBUF + 2]
        sems = scr[NBUF + 3:]
        wid = lax.axis_index("c") * 16 + lax.axis_index("s")
        pltpu.sync_copy(x_hbm.at[pl.ds(wid * (2 * EPW), 2 * EPW)], idx_v)
        pltpu.sync_copy(we_hbm, we_v)
        pltpu.sync_copy(wo_hbm, wo_v)
        lanes = lax.iota(jnp.int32, LANES)

        def issue(e, b):
            pltpu.async_copy(emb_hbm.at[idx_v.at[2 * e]],
                             rows[b].at[pl.ds(0, IDX_CHUNK)], sems[b])
            pltpu.async_copy(emb_hbm.at[idx_v.at[2 * e + 1]],
                             rows[b].at[pl.ds(IDX_CHUNK, IDX_CHUNK)], sems[b])

        def wait(b):
            pltpu.make_async_copy(emb_hbm.at[pl.ds(0, IDX_CHUNK)],
                                  rows[b].at[pl.ds(0, IDX_CHUNK)],
                                  sems[b]).wait()
            pltpu.make_async_copy(emb_hbm.at[pl.ds(0, IDX_CHUNK)],
                                  rows[b].at[pl.ds(IDX_CHUNK, IDX_CHUNK)],
                                  sems[b]).wait()

        for b in range(NBUF):
            issue(b, b)

        def round_body(r, res):
            for b in range(NBUF):
                elem = r * NBUF + b
                wait(b)

                @pl.when(r < ROUNDS - 1)
                def _():
                    issue(elem + NBUF, b)

                def body(l4, accs):
                    a0, a1 = accs
                    for kk in range(UNROLL):
                        l = l4 * UNROLL + kk
                        packed = rows[b][l, :]
                        lo = plsc.bitcast(packed << 16, jnp.float32)
                        hi = plsc.bitcast(packed & jnp.uint32(0xFFFF0000),
                                          jnp.float32)
                        a0 = a0 + lo * we_v[l, :]
                        a1 = a1 + hi * wo_v[l, :]
                    return (a0, a1)

                a0, a1 = lax.fori_loop(
                    0, MAX_LEN // UNROLL, body,
                    (jnp.zeros(LANES, jnp.float32),
                     jnp.zeros(LANES, jnp.float32)))
                s = jnp.sum(a0 + a1)
                res = jnp.where(lanes == (r % 2) * NBUF + b, s, res)

            @pl.when(r % 2 == 1)
            def _():
                off = pl.multiple_of((r // 2) * (2 * NBUF), 2 * NBUF)
                acc_v[pl.ds(off, LANES)] = res

            return jnp.where(r % 2 == 1, jnp.zeros_like(res), res)

        lax.fori_loop(0, ROUNDS, round_body, jnp.zeros(LANES, jnp.float32))
        pltpu.sync_copy(acc_v, out_hbm.at[pl.ds(wid * EPW, EPW)])

    return k(xr4, emb_rows, w_even, w_odd)


@jax.jit
def kernel(x, emb_table, W, b):
    xr = x.astype(jnp.int32).reshape(2 * BATCH, IDX_CHUNK)
    w2 = W.reshape(MAX_LEN, EMBED_DIM)
    w_lo = w2[:, :EMBED_DIM // 2]
    w_hi = w2[:, EMBED_DIM // 2:]
    emb_packed = _tc_format_bf16(emb_table.T)
    emb_view = emb_packed.reshape(emb_table.shape[0], EMBED_DIM // 2)
    out = _sc_classify(xr, emb_view, w_lo, w_hi)
    return out.reshape(BATCH, 1) + b


# speedup vs baseline: 3.1901x; 1.2859x over previous
import dataclasses
import functools

import jax
import jax.numpy as jnp
from jax import lax
from jax.experimental import pallas as pl
from jax.experimental.pallas import tpu as pltpu
from jax.experimental.pallas import tpu_sc as plsc

BATCH = 4096
MAX_LEN = 200
EMBED_DIM = 32
NUM_WORKERS = 32
EPW = BATCH // NUM_WORKERS
IDX_CHUNK = 100
LANES = 16
NBUF = 8
ROUNDS = EPW // NBUF
UNROLL = 4


TRANS_BLOCK = 16384


def _tc_transpose(emb_t):
    vocab = emb_t.shape[1]
    grid = pl.cdiv(vocab, TRANS_BLOCK)
    packed_rows = TRANS_BLOCK // 4

    def body(x_ref, o_ref, t_ref):
        t_ref[...] = x_ref[...].T
        o_ref[...] = jnp.concatenate(
            [t_ref[pl.Slice(k, packed_rows, 4), :] for k in range(4)], axis=1)

    packed = pl.pallas_call(
        body,
        grid=(grid,),
        in_specs=[pl.BlockSpec((EMBED_DIM, TRANS_BLOCK), lambda i: (0, i))],
        out_specs=pl.BlockSpec((packed_rows, 4 * EMBED_DIM), lambda i: (i, 0)),
        out_shape=jax.ShapeDtypeStruct((vocab // 4, 4 * EMBED_DIM), jnp.float32),
        scratch_shapes=[pltpu.VMEM((TRANS_BLOCK, EMBED_DIM), jnp.float32)],
    )(emb_t)
    return packed.reshape(vocab, EMBED_DIM)


def _sc_classify(xr, emb_table, w2):
    mesh = plsc.VectorSubcoreMesh(core_axis_name="c", subcore_axis_name="s")
    cp = pltpu.CompilerParams()
    for field, val in (("needs_layout_passes", False),
                       ("use_tc_tiling_on_sc", False)):
        if field in pltpu.CompilerParams.__dataclass_fields__:
            cp = dataclasses.replace(cp, **{field: val})

    @functools.partial(
        pl.kernel,
        out_type=jax.ShapeDtypeStruct((BATCH,), jnp.float32),
        mesh=mesh,
        compiler_params=cp,
        scratch_types=(
            [pltpu.VMEM((2 * EPW, IDX_CHUNK), jnp.int32)]
            + [pltpu.VMEM((MAX_LEN, EMBED_DIM), jnp.float32)
               for _ in range(NBUF)]
            + [pltpu.VMEM((MAX_LEN, EMBED_DIM), jnp.float32)]
            + [pltpu.VMEM((EPW,), jnp.float32)]
            + [pltpu.SemaphoreType.DMA for _ in range(NBUF)]
        ),
    )
    def k(x_hbm, emb_hbm, w_hbm, out_hbm, idx_v, *scr):
        rows = scr[:NBUF]
        w_v = scr[NBUF]
        acc_v = scr[NBUF + 1]
        sems = scr[NBUF + 2:]
        wid = lax.axis_index("c") * 16 + lax.axis_index("s")
        pltpu.sync_copy(x_hbm.at[pl.ds(wid * (2 * EPW), 2 * EPW)], idx_v)
        pltpu.sync_copy(w_hbm, w_v)
        lanes = lax.iota(jnp.int32, LANES)

        def issue(e, b):
            pltpu.async_copy(emb_hbm.at[idx_v.at[2 * e]],
                             rows[b].at[pl.ds(0, IDX_CHUNK)], sems[b])
            pltpu.async_copy(emb_hbm.at[idx_v.at[2 * e + 1]],
                             rows[b].at[pl.ds(IDX_CHUNK, IDX_CHUNK)], sems[b])

        def wait(b):
            pltpu.make_async_copy(emb_hbm.at[pl.ds(0, IDX_CHUNK)],
                                  rows[b].at[pl.ds(0, IDX_CHUNK)],
                                  sems[b]).wait()
            pltpu.make_async_copy(emb_hbm.at[pl.ds(0, IDX_CHUNK)],
                                  rows[b].at[pl.ds(IDX_CHUNK, IDX_CHUNK)],
                                  sems[b]).wait()

        for b in range(NBUF):
            issue(b, b)

        def round_body(r, res):
            for b in range(NBUF):
                elem = r * NBUF + b
                wait(b)

                @pl.when(r < ROUNDS - 1)
                def _():
                    issue(elem + NBUF, b)

                def body(l4, accs):
                    a0, a1 = accs
                    for kk in range(UNROLL):
                        l = l4 * UNROLL + kk
                        a0 = a0 + rows[b][l, pl.ds(0, LANES)] * w_v[l, pl.ds(0, LANES)]
                        a1 = a1 + rows[b][l, pl.ds(LANES, LANES)] * w_v[l, pl.ds(LANES, LANES)]
                    return (a0, a1)

                a0, a1 = lax.fori_loop(
                    0, MAX_LEN // UNROLL, body,
                    (jnp.zeros(LANES, jnp.float32),
                     jnp.zeros(LANES, jnp.float32)))
                s = jnp.sum(a0 + a1)
                res = jnp.where(lanes == (r % 2) * NBUF + b, s, res)

            @pl.when(r % 2 == 1)
            def _():
                off = pl.multiple_of((r // 2) * (2 * NBUF), 2 * NBUF)
                acc_v[pl.ds(off, LANES)] = res

            return jnp.where(r % 2 == 1, jnp.zeros_like(res), res)

        lax.fori_loop(0, ROUNDS, round_body, jnp.zeros(LANES, jnp.float32))
        pltpu.sync_copy(acc_v, out_hbm.at[pl.ds(wid * EPW, EPW)])

    return k(xr, emb_table, w2)


@jax.jit
def kernel(x, emb_table, W, b):
    xr = x.reshape(2 * BATCH, IDX_CHUNK).astype(jnp.int32)
    w2 = W.reshape(MAX_LEN, EMBED_DIM)
    emb_rm = _tc_transpose(emb_table.T)
    out = _sc_classify(xr, emb_rm, w2)
    return out.reshape(BATCH, 1) + b


# speedup vs baseline: 3.2114x; 1.0067x over previous
import dataclasses
import functools

import jax
import jax.numpy as jnp
from jax import lax
from jax.experimental import pallas as pl
from jax.experimental.pallas import tpu as pltpu
from jax.experimental.pallas import tpu_sc as plsc

BATCH = 4096
MAX_LEN = 200
EMBED_DIM = 32
NUM_WORKERS = 32
EPW = BATCH // NUM_WORKERS
IDX_CHUNK = 100
LANES = 16
NBUF = 8
ROUNDS = EPW // NBUF
UNROLL = 4


TRANS_BLOCK = 32768


def _tc_transpose(emb_t):
    vocab = emb_t.shape[1]
    grid = pl.cdiv(vocab, TRANS_BLOCK)
    packed_rows = TRANS_BLOCK // 4

    def body(x_ref, o_ref, t_ref):
        t_ref[...] = x_ref[...].T
        o_ref[...] = jnp.concatenate(
            [t_ref[pl.Slice(k, packed_rows, 4), :] for k in range(4)], axis=1)

    packed = pl.pallas_call(
        body,
        grid=(grid,),
        in_specs=[pl.BlockSpec((EMBED_DIM, TRANS_BLOCK), lambda i: (0, i))],
        out_specs=pl.BlockSpec((packed_rows, 4 * EMBED_DIM), lambda i: (i, 0)),
        out_shape=jax.ShapeDtypeStruct((vocab // 4, 4 * EMBED_DIM), jnp.float32),
        scratch_shapes=[pltpu.VMEM((TRANS_BLOCK, EMBED_DIM), jnp.float32)],
    )(emb_t)
    return packed.reshape(vocab, EMBED_DIM)


def _sc_classify(xr, emb_table, w2):
    mesh = plsc.VectorSubcoreMesh(core_axis_name="c", subcore_axis_name="s")
    cp = pltpu.CompilerParams()
    for field, val in (("needs_layout_passes", False),
                       ("use_tc_tiling_on_sc", False)):
        if field in pltpu.CompilerParams.__dataclass_fields__:
            cp = dataclasses.replace(cp, **{field: val})

    @functools.partial(
        pl.kernel,
        out_type=jax.ShapeDtypeStruct((BATCH,), jnp.float32),
        mesh=mesh,
        compiler_params=cp,
        scratch_types=(
            [pltpu.VMEM((2 * EPW, IDX_CHUNK), jnp.int32)]
            + [pltpu.VMEM((MAX_LEN, EMBED_DIM), jnp.float32)
               for _ in range(NBUF)]
            + [pltpu.VMEM((MAX_LEN, EMBED_DIM), jnp.float32)]
            + [pltpu.VMEM((EPW,), jnp.float32)]
            + [pltpu.SemaphoreType.DMA for _ in range(NBUF)]
        ),
    )
    def k(x_hbm, emb_hbm, w_hbm, out_hbm, idx_v, *scr):
        rows = scr[:NBUF]
        w_v = scr[NBUF]
        acc_v = scr[NBUF + 1]
        sems = scr[NBUF + 2:]
        wid = lax.axis_index("c") * 16 + lax.axis_index("s")
        pltpu.sync_copy(x_hbm.at[pl.ds(wid * (2 * EPW), 2 * EPW)], idx_v)
        pltpu.sync_copy(w_hbm, w_v)
        lanes = lax.iota(jnp.int32, LANES)

        def issue(e, b):
            pltpu.async_copy(emb_hbm.at[idx_v.at[2 * e]],
                             rows[b].at[pl.ds(0, IDX_CHUNK)], sems[b])
            pltpu.async_copy(emb_hbm.at[idx_v.at[2 * e + 1]],
                             rows[b].at[pl.ds(IDX_CHUNK, IDX_CHUNK)], sems[b])

        def wait(b):
            pltpu.make_async_copy(emb_hbm.at[pl.ds(0, IDX_CHUNK)],
                                  rows[b].at[pl.ds(0, IDX_CHUNK)],
                                  sems[b]).wait()
            pltpu.make_async_copy(emb_hbm.at[pl.ds(0, IDX_CHUNK)],
                                  rows[b].at[pl.ds(IDX_CHUNK, IDX_CHUNK)],
                                  sems[b]).wait()

        for b in range(NBUF):
            issue(b, b)

        def round_body(r, res):
            for b in range(NBUF):
                elem = r * NBUF + b
                wait(b)

                @pl.when(r < ROUNDS - 1)
                def _():
                    issue(elem + NBUF, b)

                def body(l4, accs):
                    a0, a1 = accs
                    for kk in range(UNROLL):
                        l = l4 * UNROLL + kk
                        a0 = a0 + rows[b][l, pl.ds(0, LANES)] * w_v[l, pl.ds(0, LANES)]
                        a1 = a1 + rows[b][l, pl.ds(LANES, LANES)] * w_v[l, pl.ds(LANES, LANES)]
                    return (a0, a1)

                a0, a1 = lax.fori_loop(
                    0, MAX_LEN // UNROLL, body,
                    (jnp.zeros(LANES, jnp.float32),
                     jnp.zeros(LANES, jnp.float32)))
                s = jnp.sum(a0 + a1)
                res = jnp.where(lanes == (r % 2) * NBUF + b, s, res)

            @pl.when(r % 2 == 1)
            def _():
                off = pl.multiple_of((r // 2) * (2 * NBUF), 2 * NBUF)
                acc_v[pl.ds(off, LANES)] = res

            return jnp.where(r % 2 == 1, jnp.zeros_like(res), res)

        lax.fori_loop(0, ROUNDS, round_body, jnp.zeros(LANES, jnp.float32))
        pltpu.sync_copy(acc_v, out_hbm.at[pl.ds(wid * EPW, EPW)])

    return k(xr, emb_table, w2)


@jax.jit
def kernel(x, emb_table, W, b):
    xr = x.reshape(2 * BATCH, IDX_CHUNK).astype(jnp.int32)
    w2 = W.reshape(MAX_LEN, EMBED_DIM)
    emb_rm = _tc_transpose(emb_table.T)
    out = _sc_classify(xr, emb_rm, w2)
    return out.reshape(BATCH, 1) + b
